# baseline (speedup 1.0000x reference)
import jax
import jax.numpy as jnp
from jax.experimental import pallas as pl
from jax.experimental.pallas import tpu as pltpu

EPS = 1e-5

RB = 512
CB = 2048
N_ROWS = 4608
N_CODES = 8192
NCB = N_CODES // CB


def _body(x_ref, u_ref, es_ref, out_ref, bv_ref, bi_ref):
    j = pl.program_id(1)

    emb = es_ref[...] / jnp.maximum(u_ref[...], EPS)
    e2 = jnp.sum(emb * emb, axis=1)[None, :]
    xb = x_ref[...]
    x2 = jnp.sum(xb * xb, axis=1, keepdims=True)

    s = jax.lax.dot_general(
        xb.astype(jnp.bfloat16), emb.astype(jnp.bfloat16),
        dimension_numbers=(((1,), (1,)), ((), ())),
        preferred_element_type=jnp.float32,
    )
    d2 = jnp.maximum((x2 + e2) - 2.0 * s, 0.0)

    lmin = jnp.min(d2, axis=1, keepdims=True)
    col = jax.lax.broadcasted_iota(jnp.int32, d2.shape, 1) + j * CB
    lidx = jnp.min(jnp.where(d2 == lmin, col, jnp.int32(2**30)),
                   axis=1, keepdims=True)

    @pl.when(j == 0)
    def _():
        bv_ref[...] = lmin
        bi_ref[...] = lidx

    @pl.when(j > 0)
    def _():
        better = lmin < bv_ref[...]
        bv_ref[...] = jnp.where(better, lmin, bv_ref[...])
        bi_ref[...] = jnp.where(better, lidx, bi_ref[...])

    @pl.when(j == NCB - 1)
    def _():
        out_ref[...] = bi_ref[...]


def kernel(x, cluster_usage, embedding_sum):
    B, D, T = x.shape
    xf = jnp.transpose(x, (0, 2, 1)).reshape(B * T, D)
    usage = cluster_usage.reshape(N_CODES, 1)

    codes = pl.pallas_call(
        _body,
        grid=(N_ROWS // RB, NCB),
        in_specs=[
            pl.BlockSpec((RB, D), lambda i, j: (i, 0)),
            pl.BlockSpec((CB, 1), lambda i, j: (j, 0)),
            pl.BlockSpec((CB, D), lambda i, j: (j, 0)),
        ],
        out_specs=pl.BlockSpec((RB, 1), lambda i, j: (i, 0)),
        out_shape=jax.ShapeDtypeStruct((N_ROWS, 1), jnp.int32),
        scratch_shapes=[
            pltpu.VMEM((RB, 1), jnp.float32),
            pltpu.VMEM((RB, 1), jnp.int32),
        ],
    )(xf, usage, embedding_sum)

    return codes.reshape(B, 1, T)


# speedup vs baseline: 1.2274x; 1.2274x over previous
import jax
import jax.numpy as jnp
from jax.experimental import pallas as pl
from jax.experimental.pallas import tpu as pltpu

EPS = 1e-5

RB = 512
CB = 2048
N_ROWS = 4608
N_CODES = 8192
NCB = N_CODES // CB


def _body(x_ref, u_ref, es_ref, out_ref, ebf_ref, e2_ref, colf_ref, bv_ref, bi_ref):
    i = pl.program_id(0)
    j = pl.program_id(1)

    @pl.when(i == 0)
    def _():
        emb = es_ref[...] / jnp.maximum(u_ref[...], EPS)
        ebf_ref[pl.ds(j * CB, CB), :] = emb.astype(jnp.bfloat16)
        e2_ref[:, pl.ds(j * CB, CB)] = jnp.sum(emb * emb, axis=1)[None, :]

    @pl.when(jnp.logical_and(i == 0, j == 0))
    def _():
        colf_ref[...] = jax.lax.broadcasted_iota(
            jnp.int32, (1, CB), 1).astype(jnp.float32)

    ebf = ebf_ref[pl.ds(j * CB, CB), :]
    e2 = e2_ref[:, pl.ds(j * CB, CB)]

    xb = x_ref[...]
    x2 = jnp.sum(xb * xb, axis=1, keepdims=True)
    xbf = (xb * -2.0).astype(jnp.bfloat16)

    s = jax.lax.dot_general(
        xbf, ebf,
        dimension_numbers=(((1,), (1,)), ((), ())),
        preferred_element_type=jnp.float32,
    )
    d2 = (x2 + e2) + s

    lmin = jnp.min(d2, axis=1, keepdims=True)
    lidx = (jnp.min(jnp.where(d2 == lmin, colf_ref[...], jnp.float32(1e30)),
                    axis=1, keepdims=True)
            + (j * CB).astype(jnp.float32))

    @pl.when(j == 0)
    def _():
        bv_ref[...] = lmin
        bi_ref[...] = lidx

    @pl.when(j > 0)
    def _():
        better = lmin < bv_ref[...]
        bv_ref[...] = jnp.where(better, lmin, bv_ref[...])
        bi_ref[...] = jnp.where(better, lidx, bi_ref[...])

    @pl.when(j == NCB - 1)
    def _():
        out_ref[...] = bi_ref[...].astype(jnp.int32)


def kernel(x, cluster_usage, embedding_sum):
    B, D, T = x.shape
    xf = jnp.transpose(x, (0, 2, 1)).reshape(B * T, D)
    usage = cluster_usage.reshape(N_CODES, 1)

    codes = pl.pallas_call(
        _body,
        grid=(N_ROWS // RB, NCB),
        in_specs=[
            pl.BlockSpec((RB, D), lambda i, j: (i, 0)),
            pl.BlockSpec((CB, 1), lambda i, j: (jnp.where(i == 0, j, 0), 0)),
            pl.BlockSpec((CB, D), lambda i, j: (jnp.where(i == 0, j, 0), 0)),
        ],
        out_specs=pl.BlockSpec((RB, 1), lambda i, j: (i, 0)),
        out_shape=jax.ShapeDtypeStruct((N_ROWS, 1), jnp.int32),
        scratch_shapes=[
            pltpu.VMEM((N_CODES, 64), jnp.bfloat16),
            pltpu.VMEM((1, N_CODES), jnp.float32),
            pltpu.VMEM((1, CB), jnp.float32),
            pltpu.VMEM((RB, 1), jnp.float32),
            pltpu.VMEM((RB, 1), jnp.float32),
        ],
    )(xf, usage, embedding_sum)

    return codes.reshape(B, 1, T)
